# Initial kernel scaffold; baseline (speedup 1.0000x reference)
import jax
import jax.numpy as jnp
from jax.experimental import pallas as pl


def kernel(x, edge_index, edge_type, prev_h, emb_rel, loop_weight, skip_connect_weight, skip_connect_bias, attn_fc_w, attn_fc2_w):
    raise NotImplementedError("write your pallas kernel here")


# speedup vs baseline: 10.8530x; 10.8530x over previous
import functools

import jax
import jax.numpy as jnp
from jax import lax
from jax.experimental import pallas as pl
from jax.experimental.pallas import tpu as pltpu
from jax.experimental.pallas import tpu_sc as plsc

N = 10000
E = 320000
D = 128
NP = 10240
NW = 32
EROWS = 2560
E_PAD = EROWS * 128
ROWS_PER_W = EROWS // NW
CHUNK_ROWS = 4
CHUNK_E = CHUNK_ROWS * 128
NCHUNKS = ROWS_PER_W // CHUNK_ROWS
XE_ROWS = NP + 128


def _prepass_body(xe_ref, fc_ref, fc2_ref, o_ref):
    xe = xe_ref[...]
    fc2 = fc2_ref[...]
    cols = []
    for k in range(3):
        t = lax.dot_general(xe, fc_ref[:, 128 * k:128 * (k + 1)],
                            (((1,), (1,)), ((), ())),
                            preferred_element_type=jnp.float32)
        cols.append(lax.dot_general(t, fc2, (((1,), (1,)), ((), ())),
                                    preferred_element_type=jnp.float32))
    z = jnp.zeros((xe.shape[0], 125), jnp.float32)
    o_ref[...] = jnp.concatenate(cols + [z], axis=1)


def _combine_body(x_ref, ph_ref, u0_ref, u1_ref, s0_ref, s1_ref,
                  lw_ref, sw_ref, sb_ref, o_ref):
    xb = x_ref[...]
    ph = ph_ref[...]
    loop = jnp.dot(xb, lw_ref[...], preferred_element_type=jnp.float32)
    gate = jax.nn.sigmoid(
        jnp.dot(ph, sw_ref[...], preferred_element_type=jnp.float32)
        + sb_ref[...])
    s = s0_ref[...] + s1_ref[...]
    u = u0_ref[0] + u1_ref[0]
    s_safe = jnp.where(s > 0.0, s, 1.0)
    h = u / s_safe[:, None]
    h = h + loop
    o_ref[...] = gate * h + (1.0 - gate) * ph


def _sc_edge_body(src_h, dst_h, et_h, ssrc_h, sdst_h, srel_h, x_h,
                  u_out, s_out,
                  ssrc_l, sdst_l, srel_l, src_b, dst_b, et_b,
                  p1d, rows, shared_u, shared_s, sem):
    c = lax.axis_index("c")
    t = lax.axis_index("s")
    wid = c * 16 + t
    wrow0 = wid * ROWS_PER_W

    pltpu.sync_copy(ssrc_h, ssrc_l)
    pltpu.sync_copy(sdst_h, sdst_l)
    pltpu.sync_copy(srel_h, srel_l)

    def _zrow(i, _):
        for k in range(8):
            rows[i, pl.ds(16 * k, 16)] = jnp.zeros((16,), jnp.float32)
        return 0
    lax.fori_loop(0, 128, _zrow, 0)
    for g in range(8):
        p1d[pl.ds(16 * g, 16)] = jnp.zeros((16,), jnp.float32)
    t640 = t * 640
    for k in range(5):
        pltpu.sync_copy(rows, shared_u.at[pl.ds(t640 + 128 * k, 128)])
        pltpu.sync_copy(p1d, shared_s.at[pl.ds(t640 + 128 * k, 128)])
    plsc.subcore_barrier()

    lane = jnp.arange(16, dtype=jnp.int32)

    def _chunk(ch, _):
        base_row = wrow0 + ch
        base_edge = base_row * 128
        pltpu.sync_copy(src_h.at[pl.ds(base_row, 1)], src_b)
        pltpu.sync_copy(dst_h.at[pl.ds(base_row, 1)], dst_b)
        pltpu.sync_copy(et_h.at[pl.ds(base_row, 1)], et_b)
        gat = pltpu.async_copy(x_h.at[src_b.at[0]], rows, sem)
        for l in range(8):
            sv = src_b[0, pl.ds(16 * l, 16)]
            dv = dst_b[0, pl.ds(16 * l, 16)]
            tv = et_b[0, pl.ds(16 * l, 16)]
            a = plsc.load_gather(ssrc_l, [sv])
            b = plsc.load_gather(sdst_l, [dv])
            r = plsc.load_gather(srel_l, [tv])
            e = a + b + r
            e = jnp.maximum(e, e * 0.01)
            p = jnp.exp(e)
            eid = base_edge + 16 * l + lane
            p = jnp.where(eid < E, p, 0.0)
            p1d[pl.ds(16 * l, 16)] = p
        pltpu.sync_copy(p1d, shared_s.at[dst_b.at[0]], add=True)
        gat.wait()

        def _scale(i, _):
            for u in range(4):
                ii = i * 4 + u
                pb = plsc.load_gather(p1d, [jnp.full((16,), ii, jnp.int32)])
                for k in range(8):
                    rows[ii, pl.ds(16 * k, 16)] = (
                        rows[ii, pl.ds(16 * k, 16)] * pb)
            return 0
        lax.fori_loop(0, 32, _scale, 0)
        pltpu.sync_copy(rows, shared_u.at[dst_b.at[0]], add=True)
        return 0

    lax.fori_loop(0, ROWS_PER_W, _chunk, 0)
    plsc.subcore_barrier()

    pltpu.sync_copy(shared_u.at[pl.ds(t640, 640)],
                    u_out.at[c, pl.ds(t640, 640)])
    pltpu.sync_copy(shared_s.at[pl.ds(t640, 640)],
                    s_out.at[c, pl.ds(t640, 640)])


_sc_edge = functools.partial(
    pl.kernel,
    out_type=(jax.ShapeDtypeStruct((2, NP, D), jnp.float32),
              jax.ShapeDtypeStruct((2, NP), jnp.float32)),
    mesh=plsc.VectorSubcoreMesh(core_axis_name="c", subcore_axis_name="s"),
    compiler_params=pltpu.CompilerParams(needs_layout_passes=False),
    scratch_types=[
        pltpu.VMEM((NP,), jnp.float32),
        pltpu.VMEM((NP,), jnp.float32),
        pltpu.VMEM((128,), jnp.float32),
        pltpu.VMEM((1, 128), jnp.int32),
        pltpu.VMEM((1, 128), jnp.int32),
        pltpu.VMEM((1, 128), jnp.int32),
        pltpu.VMEM((128,), jnp.float32),
        pltpu.VMEM((128, D), jnp.float32),
        pltpu.VMEM_SHARED((NP, D), jnp.float32),
        pltpu.VMEM_SHARED((NP,), jnp.float32),
        pltpu.SemaphoreType.DMA,
    ],
)(_sc_edge_body)


def kernel(x, edge_index, edge_type, prev_h, emb_rel, loop_weight,
           skip_connect_weight, skip_connect_bias, attn_fc_w, attn_fc2_w):
    f32 = jnp.float32
    src = edge_index[0].astype(jnp.int32)
    dst = edge_index[1].astype(jnp.int32)
    et = edge_type.astype(jnp.int32)
    pad = E_PAD - E
    src2 = jnp.pad(src, (0, pad)).reshape(EROWS, 128)
    dst2 = jnp.pad(dst, (0, pad)).reshape(EROWS, 128)
    et2 = jnp.pad(et, (0, pad)).reshape(EROWS, 128)

    xe = jnp.concatenate([
        x, jnp.zeros((NP - N, D), f32),
        emb_rel, jnp.zeros((XE_ROWS - NP - emb_rel.shape[0], D), f32)], axis=0)
    out_a = pl.pallas_call(
        _prepass_body,
        grid=(3,),
        in_specs=[
            pl.BlockSpec((XE_ROWS // 3, D), lambda i: (i, 0)),
            pl.BlockSpec((D, 3 * D), lambda i: (0, 0)),
            pl.BlockSpec((1, D), lambda i: (0, 0)),
        ],
        out_specs=pl.BlockSpec((XE_ROWS // 3, D), lambda i: (i, 0)),
        out_shape=jax.ShapeDtypeStruct((XE_ROWS, D), f32),
    )(xe, attn_fc_w, attn_fc2_w)
    ssrc = out_a[:NP, 0]
    sdst = out_a[:NP, 1]
    srel = out_a[NP:NP + 128, 2]

    u_part, s_part = _sc_edge(src2, dst2, et2, ssrc, sdst, srel, x)

    x_pad = jnp.pad(x, ((0, NP - N), (0, 0)))
    ph_pad = jnp.pad(prev_h, ((0, NP - N), (0, 0)))
    blk = NP // 5
    out_c = pl.pallas_call(
        _combine_body,
        grid=(5,),
        in_specs=[
            pl.BlockSpec((blk, D), lambda i: (i, 0)),
            pl.BlockSpec((blk, D), lambda i: (i, 0)),
            pl.BlockSpec((1, blk, D), lambda i: (0, i, 0)),
            pl.BlockSpec((1, blk, D), lambda i: (1, i, 0)),
            pl.BlockSpec((blk,), lambda i: (i,)),
            pl.BlockSpec((blk,), lambda i: (i,)),
            pl.BlockSpec((D, D), lambda i: (0, 0)),
            pl.BlockSpec((D, D), lambda i: (0, 0)),
            pl.BlockSpec((1, D), lambda i: (0, 0)),
        ],
        out_specs=pl.BlockSpec((blk, D), lambda i: (i, 0)),
        out_shape=jax.ShapeDtypeStruct((NP, D), f32),
    )(x_pad, ph_pad, u_part, u_part, s_part[0], s_part[1],
      loop_weight, skip_connect_weight, skip_connect_bias.reshape(1, D))
    return out_c[:N]


# speedup vs baseline: 12.0994x; 1.1149x over previous
import functools

import jax
import jax.numpy as jnp
from jax import lax
from jax.experimental import pallas as pl
from jax.experimental.pallas import tpu as pltpu
from jax.experimental.pallas import tpu_sc as plsc

N = 10000
E = 320000
D = 128
NP = 10240
NW = 32
EROWS = 2560
E_PAD = EROWS * 128
ROWS_PER_W = EROWS // NW
XE_ROWS = NP + 128


def _prepass_body(xe_ref, fc_ref, fc2_ref, o_ref):
    xe = xe_ref[...]
    fc2 = fc2_ref[...]
    cols = []
    for k in range(3):
        t = lax.dot_general(xe, fc_ref[:, 128 * k:128 * (k + 1)],
                            (((1,), (1,)), ((), ())),
                            preferred_element_type=jnp.float32)
        cols.append(lax.dot_general(t, fc2, (((1,), (1,)), ((), ())),
                                    preferred_element_type=jnp.float32))
    z = jnp.zeros((xe.shape[0], 125), jnp.float32)
    o_ref[...] = jnp.concatenate(cols + [z], axis=1)


def _combine_body(x_ref, ph_ref, u0_ref, u1_ref, s0_ref, s1_ref,
                  lw_ref, sw_ref, sb_ref, o_ref):
    xb = x_ref[...]
    ph = ph_ref[...]
    loop = jnp.dot(xb, lw_ref[...], preferred_element_type=jnp.float32)
    gate = jax.nn.sigmoid(
        jnp.dot(ph, sw_ref[...], preferred_element_type=jnp.float32)
        + sb_ref[...])
    s = s0_ref[...] + s1_ref[...]
    u = u0_ref[0] + u1_ref[0]
    s_safe = jnp.where(s > 0.0, s, 1.0)
    h = u / s_safe[:, None]
    h = h + loop
    o_ref[...] = gate * h + (1.0 - gate) * ph


P_CR = 4
P_NCH = ROWS_PER_W // P_CR


def _sc_p_body(src_h, dst_h, et_h, ssrc_h, sdst_h, srel_h,
               p_out, s_out,
               ssrc_l, sdst_l, srel_l, src_b, dst_b, et_b, p_b,
               shared_s, ssem0, ssem1):
    c = lax.axis_index("c")
    t = lax.axis_index("s")
    wid = c * 16 + t
    wrow0 = wid * ROWS_PER_W
    ssem = (ssem0, ssem1)

    pltpu.sync_copy(ssrc_h, ssrc_l)
    pltpu.sync_copy(sdst_h, sdst_l)
    pltpu.sync_copy(srel_h, srel_l)

    for j in range(P_CR):
        for g in range(8):
            p_b[0, j, pl.ds(16 * g, 16)] = jnp.zeros((16,), jnp.float32)
    t640 = t * 640
    for k in range(5):
        pltpu.sync_copy(p_b.at[0, 0], shared_s.at[pl.ds(t640 + 128 * k, 128)])
    plsc.subcore_barrier()

    lane = jnp.arange(16, dtype=jnp.int32)

    def fire_stage(b, ch):
        row = wrow0 + ch * P_CR
        return [pltpu.async_copy(src_h.at[pl.ds(row, P_CR)], src_b.at[b],
                                 ssem[b]),
                pltpu.async_copy(dst_h.at[pl.ds(row, P_CR)], dst_b.at[b],
                                 ssem[b]),
                pltpu.async_copy(et_h.at[pl.ds(row, P_CR)], et_b.at[b],
                                 ssem[b])]

    def compute_p(b, ch):
        base_edge = (wrow0 + ch * P_CR) * 128
        for j in range(P_CR):
            for l in range(8):
                sv = src_b[b, j, pl.ds(16 * l, 16)]
                dv = dst_b[b, j, pl.ds(16 * l, 16)]
                tv = et_b[b, j, pl.ds(16 * l, 16)]
                a = plsc.load_gather(ssrc_l, [sv])
                bb = plsc.load_gather(sdst_l, [dv])
                r = plsc.load_gather(srel_l, [tv])
                e = a + bb + r
                e = jnp.maximum(e, e * 0.01)
                p = jnp.exp(e)
                eid = base_edge + j * 128 + 16 * l + lane
                p = jnp.where(eid < E, p, 0.0)
                p_b[b, j, pl.ds(16 * l, 16)] = p

    def do_out(b, ch):
        row = wrow0 + ch * P_CR
        for j in range(P_CR):
            pltpu.sync_copy(p_b.at[b, j], shared_s.at[dst_b.at[b, j]],
                            add=True)
        pltpu.sync_copy(p_b.at[b], p_out.at[pl.ds(row, P_CR)])

    def body(i, _):
        c0 = 2 * i
        h0 = fire_stage(0, c0)
        h1 = fire_stage(1, c0 + 1)
        for h in h0:
            h.wait()
        compute_p(0, c0)
        for h in h1:
            h.wait()
        do_out(0, c0)
        compute_p(1, c0 + 1)
        do_out(1, c0 + 1)
        return 0

    lax.fori_loop(0, P_NCH // 2, body, 0)
    plsc.subcore_barrier()
    pltpu.sync_copy(shared_s.at[pl.ds(t640, 640)],
                    s_out.at[c, pl.ds(t640, 640)])


_sc_p = functools.partial(
    pl.kernel,
    out_type=(jax.ShapeDtypeStruct((EROWS, 128), jnp.float32),
              jax.ShapeDtypeStruct((2, NP), jnp.float32)),
    mesh=plsc.VectorSubcoreMesh(core_axis_name="c", subcore_axis_name="s"),
    compiler_params=pltpu.CompilerParams(needs_layout_passes=False),
    scratch_types=[
        pltpu.VMEM((NP,), jnp.float32),
        pltpu.VMEM((NP,), jnp.float32),
        pltpu.VMEM((128,), jnp.float32),
        pltpu.VMEM((2, P_CR, 128), jnp.int32),
        pltpu.VMEM((2, P_CR, 128), jnp.int32),
        pltpu.VMEM((2, P_CR, 128), jnp.int32),
        pltpu.VMEM((2, P_CR, 128), jnp.float32),
        pltpu.VMEM_SHARED((NP,), jnp.float32),
        pltpu.SemaphoreType.DMA,
        pltpu.SemaphoreType.DMA,
    ],
)(_sc_p_body)


def _sc_u_body(src_h, dst_h, p_h, x_h,
               u_out,
               src_b, dst_b, p_b, rows, shared_u,
               gsem0, gsem1, usem0, usem1):
    c = lax.axis_index("c")
    t = lax.axis_index("s")
    wid = c * 16 + t
    wrow0 = wid * ROWS_PER_W
    gsem = (gsem0, gsem1)
    usem = (usem0, usem1)

    def _zrow(i, _):
        for k in range(8):
            rows[0, i, pl.ds(16 * k, 16)] = jnp.zeros((16,), jnp.float32)
        return 0
    lax.fori_loop(0, 128, _zrow, 0)
    t640 = t * 640
    for k in range(5):
        pltpu.sync_copy(rows.at[0],
                        shared_u.at[pl.ds(t640 + 128 * k, 128)])
    plsc.subcore_barrier()

    def stage(b, ch):
        row = wrow0 + ch
        return [pltpu.async_copy(src_h.at[pl.ds(row, 1)],
                                 src_b.at[pl.ds(b, 1)], gsem[b]),
                pltpu.async_copy(dst_h.at[pl.ds(row, 1)],
                                 dst_b.at[pl.ds(b, 1)], gsem[b]),
                pltpu.async_copy(p_h.at[pl.ds(row, 1)],
                                 p_b.at[pl.ds(b, 1)], gsem[b])]

    def scale(b):
        def _s(i, _):
            for u in range(4):
                ii = i * 4 + u
                pb = plsc.load_gather(
                    p_b, [jnp.full((16,), b, jnp.int32),
                          jnp.full((16,), ii, jnp.int32)])
                for k in range(8):
                    rows[b, ii, pl.ds(16 * k, 16)] = (
                        rows[b, ii, pl.ds(16 * k, 16)] * pb)
            return 0
        lax.fori_loop(0, 32, _s, 0)

    def body(i, _):
        c0 = 2 * i
        h0 = stage(0, c0)
        h1 = stage(1, c0 + 1)
        for h in h0:
            h.wait()
        g0 = pltpu.async_copy(x_h.at[src_b.at[0]], rows.at[0], usem[0])
        for h in h1:
            h.wait()
        g1 = pltpu.async_copy(x_h.at[src_b.at[1]], rows.at[1], usem[1])
        g0.wait()
        scale(0)
        pltpu.sync_copy(rows.at[0], shared_u.at[dst_b.at[0]], add=True)
        g1.wait()
        scale(1)
        pltpu.sync_copy(rows.at[1], shared_u.at[dst_b.at[1]], add=True)
        return 0

    lax.fori_loop(0, ROWS_PER_W // 2, body, 0)
    plsc.subcore_barrier()
    pltpu.sync_copy(shared_u.at[pl.ds(t640, 640)],
                    u_out.at[c, pl.ds(t640, 640)])


_sc_u = functools.partial(
    pl.kernel,
    out_type=jax.ShapeDtypeStruct((2, NP, D), jnp.float32),
    mesh=plsc.VectorSubcoreMesh(core_axis_name="c", subcore_axis_name="s"),
    compiler_params=pltpu.CompilerParams(needs_layout_passes=False),
    scratch_types=[
        pltpu.VMEM((2, 128), jnp.int32),
        pltpu.VMEM((2, 128), jnp.int32),
        pltpu.VMEM((2, 128), jnp.float32),
        pltpu.VMEM((2, 128, D), jnp.float32),
        pltpu.VMEM_SHARED((NP, D), jnp.float32),
        pltpu.SemaphoreType.DMA,
        pltpu.SemaphoreType.DMA,
        pltpu.SemaphoreType.DMA,
        pltpu.SemaphoreType.DMA,
    ],
)(_sc_u_body)


def kernel(x, edge_index, edge_type, prev_h, emb_rel, loop_weight,
           skip_connect_weight, skip_connect_bias, attn_fc_w, attn_fc2_w):
    f32 = jnp.float32
    src = edge_index[0].astype(jnp.int32)
    dst = edge_index[1].astype(jnp.int32)
    et = edge_type.astype(jnp.int32)
    pad = E_PAD - E
    src2 = jnp.pad(src, (0, pad)).reshape(EROWS, 128)
    dst2 = jnp.pad(dst, (0, pad)).reshape(EROWS, 128)
    et2 = jnp.pad(et, (0, pad)).reshape(EROWS, 128)

    xe = jnp.concatenate([
        x, jnp.zeros((NP - N, D), f32),
        emb_rel, jnp.zeros((XE_ROWS - NP - emb_rel.shape[0], D), f32)], axis=0)
    out_a = pl.pallas_call(
        _prepass_body,
        grid=(3,),
        in_specs=[
            pl.BlockSpec((XE_ROWS // 3, D), lambda i: (i, 0)),
            pl.BlockSpec((D, 3 * D), lambda i: (0, 0)),
            pl.BlockSpec((1, D), lambda i: (0, 0)),
        ],
        out_specs=pl.BlockSpec((XE_ROWS // 3, D), lambda i: (i, 0)),
        out_shape=jax.ShapeDtypeStruct((XE_ROWS, D), f32),
    )(xe, attn_fc_w, attn_fc2_w)
    ssrc = out_a[:NP, 0]
    sdst = out_a[:NP, 1]
    srel = out_a[NP:NP + 128, 2]

    p2, s_part = _sc_p(src2, dst2, et2, ssrc, sdst, srel)
    u_pad = _sc_u(src2, dst2, p2, x)

    x_pad = jnp.pad(x, ((0, NP - N), (0, 0)))
    ph_pad = jnp.pad(prev_h, ((0, NP - N), (0, 0)))
    blk = NP // 5
    out_c = pl.pallas_call(
        _combine_body,
        grid=(5,),
        in_specs=[
            pl.BlockSpec((blk, D), lambda i: (i, 0)),
            pl.BlockSpec((blk, D), lambda i: (i, 0)),
            pl.BlockSpec((1, blk, D), lambda i: (0, i, 0)),
            pl.BlockSpec((1, blk, D), lambda i: (1, i, 0)),
            pl.BlockSpec((blk,), lambda i: (i,)),
            pl.BlockSpec((blk,), lambda i: (i,)),
            pl.BlockSpec((D, D), lambda i: (0, 0)),
            pl.BlockSpec((D, D), lambda i: (0, 0)),
            pl.BlockSpec((1, D), lambda i: (0, 0)),
        ],
        out_specs=pl.BlockSpec((blk, D), lambda i: (i, 0)),
        out_shape=jax.ShapeDtypeStruct((NP, D), f32),
    )(x_pad, ph_pad, u_pad, u_pad, s_part[0], s_part[1],
      loop_weight, skip_connect_weight, skip_connect_bias.reshape(1, D))
    return out_c[:N]


# speedup vs baseline: 12.1817x; 1.0068x over previous
import functools

import jax
import jax.numpy as jnp
from jax import lax
from jax.experimental import pallas as pl
from jax.experimental.pallas import tpu as pltpu
from jax.experimental.pallas import tpu_sc as plsc

N = 10000
E = 320000
D = 128
NP = 10240
NW = 32
EROWS = 2560
E_PAD = EROWS * 128
ROWS_PER_W = EROWS // NW
XE_ROWS = NP + 128


def _prepass_body(xe_ref, fc_ref, fc2_ref, o_ref):
    xe = xe_ref[...]
    fc2 = fc2_ref[...]
    cols = []
    for k in range(3):
        t = lax.dot_general(xe, fc_ref[:, 128 * k:128 * (k + 1)],
                            (((1,), (1,)), ((), ())),
                            preferred_element_type=jnp.float32)
        cols.append(lax.dot_general(t, fc2, (((1,), (1,)), ((), ())),
                                    preferred_element_type=jnp.float32))
    z = jnp.zeros((xe.shape[0], 125), jnp.float32)
    o_ref[...] = jnp.concatenate(cols + [z], axis=1)


def _combine_body(x_ref, ph_ref, u0_ref, u1_ref, s0_ref, s1_ref,
                  lw_ref, sw_ref, sb_ref, o_ref):
    xb = x_ref[...]
    ph = ph_ref[...]
    loop = jnp.dot(xb, lw_ref[...], preferred_element_type=jnp.float32)
    gate = jax.nn.sigmoid(
        jnp.dot(ph, sw_ref[...], preferred_element_type=jnp.float32)
        + sb_ref[...])
    s = s0_ref[...] + s1_ref[...]
    u = u0_ref[0] + u1_ref[0]
    s_safe = jnp.where(s > 0.0, s, 1.0)
    h = u / s_safe[:, None]
    h = h + loop
    o_ref[...] = gate * h + (1.0 - gate) * ph


P_CR = 4
P_NCH = ROWS_PER_W // P_CR


def _sc_p_body(src_h, dst_h, et_h, ssrc_h, sdst_h, srel_h,
               p_out, s_out,
               ssrc_l, sdst_l, srel_l, src_b, dst_b, et_b, p_b,
               shared_s, ssem0, ssem1):
    c = lax.axis_index("c")
    t = lax.axis_index("s")
    wid = c * 16 + t
    wrow0 = wid * ROWS_PER_W
    ssem = (ssem0, ssem1)

    pltpu.sync_copy(ssrc_h, ssrc_l)
    pltpu.sync_copy(sdst_h, sdst_l)
    pltpu.sync_copy(srel_h, srel_l)

    for j in range(P_CR):
        for g in range(8):
            p_b[0, j, pl.ds(16 * g, 16)] = jnp.zeros((16,), jnp.float32)
    t640 = t * 640
    for k in range(5):
        pltpu.sync_copy(p_b.at[0, 0], shared_s.at[pl.ds(t640 + 128 * k, 128)])
    plsc.subcore_barrier()

    lane = jnp.arange(16, dtype=jnp.int32)

    def fire_stage(b, ch):
        row = wrow0 + ch * P_CR
        return [pltpu.async_copy(src_h.at[pl.ds(row, P_CR)], src_b.at[b],
                                 ssem[b]),
                pltpu.async_copy(dst_h.at[pl.ds(row, P_CR)], dst_b.at[b],
                                 ssem[b]),
                pltpu.async_copy(et_h.at[pl.ds(row, P_CR)], et_b.at[b],
                                 ssem[b])]

    def compute_p(b, ch):
        base_edge = (wrow0 + ch * P_CR) * 128
        for j in range(P_CR):
            for l in range(8):
                sv = src_b[b, j, pl.ds(16 * l, 16)]
                dv = dst_b[b, j, pl.ds(16 * l, 16)]
                tv = et_b[b, j, pl.ds(16 * l, 16)]
                a = plsc.load_gather(ssrc_l, [sv])
                bb = plsc.load_gather(sdst_l, [dv])
                r = plsc.load_gather(srel_l, [tv])
                e = a + bb + r
                e = jnp.maximum(e, e * 0.01)
                p = jnp.exp(e)
                eid = base_edge + j * 128 + 16 * l + lane
                p = jnp.where(eid < E, p, 0.0)
                p_b[b, j, pl.ds(16 * l, 16)] = p

    def do_out(b, ch):
        row = wrow0 + ch * P_CR
        for j in range(P_CR):
            pltpu.sync_copy(p_b.at[b, j], shared_s.at[dst_b.at[b, j]],
                            add=True)
        pltpu.sync_copy(p_b.at[b], p_out.at[pl.ds(row, P_CR)])

    def body(i, _):
        c0 = 2 * i
        h0 = fire_stage(0, c0)
        h1 = fire_stage(1, c0 + 1)
        for h in h0:
            h.wait()
        compute_p(0, c0)
        for h in h1:
            h.wait()
        do_out(0, c0)
        compute_p(1, c0 + 1)
        do_out(1, c0 + 1)
        return 0

    lax.fori_loop(0, P_NCH // 2, body, 0)
    plsc.subcore_barrier()
    pltpu.sync_copy(shared_s.at[pl.ds(t640, 640)],
                    s_out.at[c, pl.ds(t640, 640)])


_sc_p = functools.partial(
    pl.kernel,
    out_type=(jax.ShapeDtypeStruct((EROWS, 128), jnp.float32),
              jax.ShapeDtypeStruct((2, NP), jnp.float32)),
    mesh=plsc.VectorSubcoreMesh(core_axis_name="c", subcore_axis_name="s"),
    compiler_params=pltpu.CompilerParams(needs_layout_passes=False),
    scratch_types=[
        pltpu.VMEM((NP,), jnp.float32),
        pltpu.VMEM((NP,), jnp.float32),
        pltpu.VMEM((128,), jnp.float32),
        pltpu.VMEM((2, P_CR, 128), jnp.int32),
        pltpu.VMEM((2, P_CR, 128), jnp.int32),
        pltpu.VMEM((2, P_CR, 128), jnp.int32),
        pltpu.VMEM((2, P_CR, 128), jnp.float32),
        pltpu.VMEM_SHARED((NP,), jnp.float32),
        pltpu.SemaphoreType.DMA,
        pltpu.SemaphoreType.DMA,
    ],
)(_sc_p_body)


def _sc_u_body(src_h, dst_h, p_h, x_h,
               u_out,
               src_b, dst_b, p_b, rows, shared_u,
               gsem0, gsem1, usem0, usem1, vsem0, vsem1):
    c = lax.axis_index("c")
    t = lax.axis_index("s")
    wid = c * 16 + t
    wrow0 = wid * ROWS_PER_W
    gsem = (gsem0, gsem1)
    usem = (usem0, usem1)
    vsem = (vsem0, vsem1)

    def _zrow(i, _):
        for k in range(8):
            rows[0, i, pl.ds(16 * k, 16)] = jnp.zeros((16,), jnp.float32)
        return 0
    lax.fori_loop(0, 128, _zrow, 0)
    t640 = t * 640
    for k in range(5):
        pltpu.sync_copy(rows.at[0],
                        shared_u.at[pl.ds(t640 + 128 * k, 128)])
    plsc.subcore_barrier()

    def stage(b, ch):
        row = wrow0 + ch
        return [pltpu.async_copy(src_h.at[pl.ds(row, 1)],
                                 src_b.at[pl.ds(b, 1)], gsem[b]),
                pltpu.async_copy(dst_h.at[pl.ds(row, 1)],
                                 dst_b.at[pl.ds(b, 1)], gsem[b]),
                pltpu.async_copy(p_h.at[pl.ds(row, 1)],
                                 p_b.at[pl.ds(b, 1)], gsem[b])]

    def scale(b):
        def _s(i, _):
            for u in range(4):
                ii = i * 4 + u
                pb = plsc.load_gather(
                    p_b, [jnp.full((16,), b, jnp.int32),
                          jnp.full((16,), ii, jnp.int32)])
                for k in range(8):
                    rows[b, ii, pl.ds(16 * k, 16)] = (
                        rows[b, ii, pl.ds(16 * k, 16)] * pb)
            return 0
        lax.fori_loop(0, 32, _s, 0)

    def fire_gathers(b):
        return [pltpu.async_copy(x_h.at[src_b.at[b, pl.ds(0, 64)]],
                                 rows.at[b, pl.ds(0, 64)], usem[b]),
                pltpu.async_copy(x_h.at[src_b.at[b, pl.ds(64, 64)]],
                                 rows.at[b, pl.ds(64, 64)], vsem[b])]

    def body(i, _):
        c0 = 2 * i
        h0 = stage(0, c0)
        h1 = stage(1, c0 + 1)
        for h in h0:
            h.wait()
        g0 = fire_gathers(0)
        for h in h1:
            h.wait()
        g1 = fire_gathers(1)
        for g in g0:
            g.wait()
        scale(0)
        pltpu.sync_copy(rows.at[0], shared_u.at[dst_b.at[0]], add=True)
        for g in g1:
            g.wait()
        scale(1)
        pltpu.sync_copy(rows.at[1], shared_u.at[dst_b.at[1]], add=True)
        return 0

    lax.fori_loop(0, ROWS_PER_W // 2, body, 0)
    plsc.subcore_barrier()
    pltpu.sync_copy(shared_u.at[pl.ds(t640, 640)],
                    u_out.at[c, pl.ds(t640, 640)])


_sc_u = functools.partial(
    pl.kernel,
    out_type=jax.ShapeDtypeStruct((2, NP, D), jnp.float32),
    mesh=plsc.VectorSubcoreMesh(core_axis_name="c", subcore_axis_name="s"),
    compiler_params=pltpu.CompilerParams(needs_layout_passes=False),
    scratch_types=[
        pltpu.VMEM((2, 128), jnp.int32),
        pltpu.VMEM((2, 128), jnp.int32),
        pltpu.VMEM((2, 128), jnp.float32),
        pltpu.VMEM((2, 128, D), jnp.float32),
        pltpu.VMEM_SHARED((NP, D), jnp.float32),
        pltpu.SemaphoreType.DMA,
        pltpu.SemaphoreType.DMA,
        pltpu.SemaphoreType.DMA,
        pltpu.SemaphoreType.DMA,
        pltpu.SemaphoreType.DMA,
        pltpu.SemaphoreType.DMA,
    ],
)(_sc_u_body)


def kernel(x, edge_index, edge_type, prev_h, emb_rel, loop_weight,
           skip_connect_weight, skip_connect_bias, attn_fc_w, attn_fc2_w):
    f32 = jnp.float32
    src = edge_index[0].astype(jnp.int32)
    dst = edge_index[1].astype(jnp.int32)
    et = edge_type.astype(jnp.int32)
    pad = E_PAD - E
    src2 = jnp.pad(src, (0, pad)).reshape(EROWS, 128)
    dst2 = jnp.pad(dst, (0, pad)).reshape(EROWS, 128)
    et2 = jnp.pad(et, (0, pad)).reshape(EROWS, 128)

    xe = jnp.concatenate([
        x, jnp.zeros((NP - N, D), f32),
        emb_rel, jnp.zeros((XE_ROWS - NP - emb_rel.shape[0], D), f32)], axis=0)
    out_a = pl.pallas_call(
        _prepass_body,
        grid=(3,),
        in_specs=[
            pl.BlockSpec((XE_ROWS // 3, D), lambda i: (i, 0)),
            pl.BlockSpec((D, 3 * D), lambda i: (0, 0)),
            pl.BlockSpec((1, D), lambda i: (0, 0)),
        ],
        out_specs=pl.BlockSpec((XE_ROWS // 3, D), lambda i: (i, 0)),
        out_shape=jax.ShapeDtypeStruct((XE_ROWS, D), f32),
    )(xe, attn_fc_w, attn_fc2_w)
    ssrc = out_a[:NP, 0]
    sdst = out_a[:NP, 1]
    srel = out_a[NP:NP + 128, 2]

    p2, s_part = _sc_p(src2, dst2, et2, ssrc, sdst, srel)
    u_pad = _sc_u(src2, dst2, p2, x)

    x_pad = jnp.pad(x, ((0, NP - N), (0, 0)))
    ph_pad = jnp.pad(prev_h, ((0, NP - N), (0, 0)))
    blk = NP // 5
    out_c = pl.pallas_call(
        _combine_body,
        grid=(5,),
        in_specs=[
            pl.BlockSpec((blk, D), lambda i: (i, 0)),
            pl.BlockSpec((blk, D), lambda i: (i, 0)),
            pl.BlockSpec((1, blk, D), lambda i: (0, i, 0)),
            pl.BlockSpec((1, blk, D), lambda i: (1, i, 0)),
            pl.BlockSpec((blk,), lambda i: (i,)),
            pl.BlockSpec((blk,), lambda i: (i,)),
            pl.BlockSpec((D, D), lambda i: (0, 0)),
            pl.BlockSpec((D, D), lambda i: (0, 0)),
            pl.BlockSpec((1, D), lambda i: (0, 0)),
        ],
        out_specs=pl.BlockSpec((blk, D), lambda i: (i, 0)),
        out_shape=jax.ShapeDtypeStruct((NP, D), f32),
    )(x_pad, ph_pad, u_pad, u_pad, s_part[0], s_part[1],
      loop_weight, skip_connect_weight, skip_connect_bias.reshape(1, D))
    return out_c[:N]
